# baseline (speedup 1.0000x reference)
import functools

import jax
import jax.numpy as jnp
from jax import lax
from jax.experimental import pallas as pl
from jax.experimental.pallas import tpu as pltpu
from jax.experimental.pallas import tpu_sc as plsc

E = 64
C = 8
K = 16
L = 16
NC = 2
NS = 16
NW = NC * NS
UNROLL = 4


def _build_sc_kernel(B: int):
  R = B // NW
  mesh = plsc.VectorSubcoreMesh(core_axis_name="c", subcore_axis_name="s")

  @functools.partial(
      pl.kernel,
      out_type=(
          jax.ShapeDtypeStruct((B * K,), jnp.int32),
          jax.ShapeDtypeStruct((B * K,), jnp.float32),
      ),
      mesh=mesh,
      compiler_params=pltpu.CompilerParams(needs_layout_passes=False),
      scratch_types=[
          pltpu.VMEM((E * E,), jnp.float32),
          pltpu.VMEM((R * C + L,), jnp.int32),
          pltpu.VMEM((R * C + L,), jnp.float32),
          pltpu.VMEM((L,), jnp.int32),
          pltpu.VMEM((L,), jnp.float32),
          pltpu.VMEM((R * K,), jnp.int32),
          pltpu.VMEM((R * K,), jnp.float32),
      ],
  )
  def sc_kernel(scores_hbm, cooc_hbm, ids_hbm, dvi_hbm, dvf_hbm,
                oi_hbm, os_hbm,
                cooc_v, ids_v, sc_v, di_v, df_v, oi_v, os_v):
    wid = lax.axis_index("s") * NC + lax.axis_index("c")
    base = wid * R
    pltpu.sync_copy(cooc_hbm, cooc_v)
    pltpu.sync_copy(ids_hbm.at[pl.ds(base * C, R * C)],
                    ids_v.at[pl.ds(0, R * C)])
    pltpu.sync_copy(scores_hbm.at[pl.ds(base * C, R * C)],
                    sc_v.at[pl.ds(0, R * C)])
    pltpu.sync_copy(dvi_hbm, di_v)
    pltpu.sync_copy(dvf_hbm, df_v)

    lane = lax.iota(jnp.int32, L)
    mask_lo = lane < C
    neg_inf = jnp.full((L,), -jnp.inf, jnp.float32)
    vals = [lane + j * L for j in range(E // L)]
    di = di_v[...]
    df = df_v[...]
    ids_v[pl.ds(R * C, L)] = jnp.zeros((L,), jnp.int32)

    @plsc.parallel_loop(0, R, 1, unroll=UNROLL)
    def row_body(r):
      o8 = r * C
      ids16 = ids_v[pl.ds(o8, L)]
      s16 = sc_v[pl.ds(o8, L)]
      accs = None
      cids = [ids16[i] for i in range(C)]
      for i in range(C):
        s = s16[i]
        cb = cids[i] * E
        rows = [cooc_v[pl.ds(cb + j * L, L)] for j in range(E // L)]
        if accs is None:
          accs = [s * rj for rj in rows]
        else:
          accs = [a + s * rj for a, rj in zip(accs, rows)]
      for i in range(C):
        accs = [jnp.where(vj == cids[i], neg_inf, aj)
                for vj, aj in zip(vals, accs)]
      s0k, s0v = plsc.sort_key_val(accs[0], vals[0], descending=True)
      s1k, s1v = plsc.sort_key_val(accs[1], vals[1], descending=False)
      s2k, s2v = plsc.sort_key_val(accs[2], vals[2], descending=True)
      s3k, s3v = plsc.sort_key_val(accs[3], vals[3], descending=False)
      t01k, t01v = plsc.sort_key_val(jnp.where(mask_lo, s0k, s1k),
                                     jnp.where(mask_lo, s0v, s1v),
                                     descending=True)
      t23k, t23v = plsc.sort_key_val(jnp.where(mask_lo, s2k, s3k),
                                     jnp.where(mask_lo, s2v, s3v),
                                     descending=False)
      fk, fv = plsc.sort_key_val(jnp.where(mask_lo, t01k, t23k),
                                 jnp.where(mask_lo, t01v, t23v),
                                 descending=True)
      rk = r * K
      oi_v[pl.ds(rk, L)] = ids16 + di
      os_v[pl.ds(rk, L)] = s16 + df
      hi_idx = lane + (rk + C)
      plsc.store_scatter(oi_v, [hi_idx], fv + di, mask=mask_lo)
      plsc.store_scatter(os_v, [hi_idx], fk + df, mask=mask_lo)

    pltpu.sync_copy(oi_v, oi_hbm.at[pl.ds(base * K, R * K)])
    pltpu.sync_copy(os_v, os_hbm.at[pl.ds(base * K, R * K)])

  return sc_kernel


@functools.cache
def _get_sc_kernel(B: int):
  return _build_sc_kernel(B)


def kernel(candidate_scores, cooccurrence, candidate_ids, target_size):
  B, _ = candidate_ids.shape
  delta_i = jnp.asarray(target_size, jnp.int32) - K
  dvi = jnp.full((L,), delta_i, jnp.int32)
  dvf = jnp.full((L,), delta_i.astype(jnp.float32), jnp.float32)
  oi, os_ = _get_sc_kernel(B)(
      candidate_scores.reshape(-1),
      cooccurrence.reshape(-1),
      candidate_ids.reshape(-1),
      dvi,
      dvf,
  )
  return oi.reshape(B, K), os_.reshape(B, K)


# speedup vs baseline: 1.2594x; 1.2594x over previous
import functools

import jax
import jax.numpy as jnp
from jax import lax
from jax.experimental import pallas as pl
from jax.experimental.pallas import tpu as pltpu
from jax.experimental.pallas import tpu_sc as plsc

E = 64
C = 8
K = 16
L = 16
NC = 2
NS = 16
NW = NC * NS
UNROLL = 4


def _build_sc_kernel(B: int):
  R = B // NW
  mesh = plsc.VectorSubcoreMesh(core_axis_name="c", subcore_axis_name="s")

  @functools.partial(
      pl.kernel,
      out_type=(
          jax.ShapeDtypeStruct((B * K,), jnp.int32),
          jax.ShapeDtypeStruct((B * K,), jnp.float32),
      ),
      mesh=mesh,
      compiler_params=pltpu.CompilerParams(needs_layout_passes=False),
      scratch_types=[
          pltpu.VMEM((E * E,), jnp.float32),
          pltpu.VMEM((R * C + L,), jnp.int32),
          pltpu.VMEM((R * C + L,), jnp.float32),
          pltpu.VMEM((L,), jnp.int32),
          pltpu.VMEM((L,), jnp.float32),
          pltpu.VMEM((R * K,), jnp.int32),
          pltpu.VMEM((R * K,), jnp.float32),
      ],
  )
  def sc_kernel(scores_hbm, cooc_hbm, ids_hbm, dvi_hbm, dvf_hbm,
                oi_hbm, os_hbm,
                cooc_v, ids_v, sc_v, di_v, df_v, oi_v, os_v):
    wid = lax.axis_index("s") * NC + lax.axis_index("c")
    base = wid * R
    pltpu.sync_copy(cooc_hbm, cooc_v)
    pltpu.sync_copy(ids_hbm.at[pl.ds(base * C, R * C)],
                    ids_v.at[pl.ds(0, R * C)])
    pltpu.sync_copy(scores_hbm.at[pl.ds(base * C, R * C)],
                    sc_v.at[pl.ds(0, R * C)])
    pltpu.sync_copy(dvi_hbm, di_v)
    pltpu.sync_copy(dvf_hbm, df_v)

    lane = lax.iota(jnp.int32, L)
    mask_lo = lane < C
    neg_inf = jnp.full((L,), -jnp.inf, jnp.float32)
    vals = [lane + j * L for j in range(E // L)]
    di = di_v[...]
    df = df_v[...]
    ids_v[pl.ds(R * C, L)] = jnp.zeros((L,), jnp.int32)

    def row_body(r, carry):
      o8 = r * C
      ids16 = ids_v[pl.ds(o8, L)]
      s16 = sc_v[pl.ds(o8, L)]
      accs = None
      cids = [ids16[i] for i in range(C)]
      for i in range(C):
        s = s16[i]
        cb = cids[i] * E
        rows = [cooc_v[pl.ds(cb + j * L, L)] for j in range(E // L)]
        if accs is None:
          accs = [s * rj for rj in rows]
        else:
          accs = [a + s * rj for a, rj in zip(accs, rows)]
      for i in range(C):
        accs = [jnp.where(vj == cids[i], neg_inf, aj)
                for vj, aj in zip(vals, accs)]
      s0k, s0v = plsc.sort_key_val(accs[0], vals[0], descending=True)
      s1k, s1v = plsc.sort_key_val(accs[1], vals[1], descending=False)
      s2k, s2v = plsc.sort_key_val(accs[2], vals[2], descending=True)
      s3k, s3v = plsc.sort_key_val(accs[3], vals[3], descending=False)
      t01k, t01v = plsc.sort_key_val(jnp.where(mask_lo, s0k, s1k),
                                     jnp.where(mask_lo, s0v, s1v),
                                     descending=True)
      t23k, t23v = plsc.sort_key_val(jnp.where(mask_lo, s2k, s3k),
                                     jnp.where(mask_lo, s2v, s3v),
                                     descending=False)
      fk, fv = plsc.sort_key_val(jnp.where(mask_lo, t01k, t23k),
                                 jnp.where(mask_lo, t01v, t23v),
                                 descending=True)
      rk = r * K
      oi_v[pl.ds(rk, L)] = ids16 + di
      os_v[pl.ds(rk, L)] = s16 + df
      hi_idx = lane + (rk + C)
      plsc.store_scatter(oi_v, [hi_idx], fv + di, mask=mask_lo)
      plsc.store_scatter(os_v, [hi_idx], fk + df, mask=mask_lo)
      return carry

    lax.fori_loop(0, R, row_body, 0)
    pltpu.sync_copy(oi_v, oi_hbm.at[pl.ds(base * K, R * K)])
    pltpu.sync_copy(os_v, os_hbm.at[pl.ds(base * K, R * K)])

  return sc_kernel


@functools.cache
def _get_sc_kernel(B: int):
  return _build_sc_kernel(B)


def kernel(candidate_scores, cooccurrence, candidate_ids, target_size):
  B, _ = candidate_ids.shape
  delta_i = jnp.asarray(target_size, jnp.int32) - K
  dvi = jnp.full((L,), delta_i, jnp.int32)
  dvf = jnp.full((L,), delta_i.astype(jnp.float32), jnp.float32)
  oi, os_ = _get_sc_kernel(B)(
      candidate_scores.reshape(-1),
      cooccurrence.reshape(-1),
      candidate_ids.reshape(-1),
      dvi,
      dvf,
  )
  return oi.reshape(B, K), os_.reshape(B, K)
